# Initial kernel scaffold; baseline (speedup 1.0000x reference)
import jax
import jax.numpy as jnp
from jax.experimental import pallas as pl


def kernel(x, edge_index, W_l1, b1, W_r1, W_l2, b2, W_r2):
    raise NotImplementedError("write your pallas kernel here")


# speedup vs baseline: 4.2629x; 4.2629x over previous
import functools

import jax
import jax.numpy as jnp
from jax import lax
from jax.experimental import pallas as pl
from jax.experimental.pallas import tpu as pltpu
from jax.experimental.pallas import tpu_sc as plsc

N = 10000
E = 320000
D = 128
N_PAD = 10240
NC, NS = 2, 16
NW = NC * NS
CH = 128
NCHUNK = 79
EW = NCHUNK * CH
E_PAD = NW * EW
PAD_DST = N
RPT = N_PAD // NS
NZ = RPT // CH
BM = 512


def _sc_body(with_counts, *refs):
    if with_counts:
        (y_hbm, eidx_hbm, zmat_hbm, zo16_hbm, widx_hbm,
         acc_out, cnt_out,
         acc_sh, cnt_sh, idx_v, rows_v, c16_v, sem) = refs
    else:
        (y_hbm, eidx_hbm, zmat_hbm,
         acc_out,
         acc_sh, idx_v, rows_v, sem) = refs

    c = lax.axis_index("c")
    s = lax.axis_index("s")
    wid = s * NC + c
    r0 = s * RPT

    pltpu.sync_copy(zmat_hbm, rows_v)

    @pl.loop(0, NZ)
    def _zero_acc(j):
        pltpu.sync_copy(rows_v, acc_sh.at[pl.ds(r0 + j * CH, CH)])

    if with_counts:
        pltpu.sync_copy(zo16_hbm.at[0], c16_v)

        @pl.loop(0, NZ)
        def _zero_cnt(j):
            pltpu.sync_copy(widx_hbm.at[s, j], idx_v)
            pltpu.sync_copy(c16_v, cnt_sh.at[idx_v.at[0]])

        pltpu.sync_copy(zo16_hbm.at[1], c16_v)

    plsc.subcore_barrier()

    @pl.loop(0, NCHUNK)
    def _edges(j):
        pltpu.sync_copy(eidx_hbm.at[wid, j], idx_v)
        pltpu.async_copy(y_hbm.at[idx_v.at[0]], rows_v, sem).wait()
        pltpu.sync_copy(rows_v, acc_sh.at[idx_v.at[1]], add=True)
        if with_counts:
            pltpu.sync_copy(c16_v, cnt_sh.at[idx_v.at[1]], add=True)

    plsc.subcore_barrier()

    @pl.loop(0, NZ)
    def _writeout(j):
        rr = r0 + j * CH
        pltpu.async_copy(acc_sh.at[pl.ds(rr, CH)], rows_v, sem).wait()
        pltpu.sync_copy(rows_v, acc_out.at[c, pl.ds(rr, CH)])

    if with_counts:

        @pl.loop(0, NZ)
        def _writeout_cnt(j):
            rr = r0 + j * CH
            pltpu.sync_copy(widx_hbm.at[s, j], idx_v)
            pltpu.async_copy(cnt_sh.at[idx_v.at[0]], c16_v, sem).wait()
            pltpu.sync_copy(c16_v, cnt_out.at[c, pl.ds(rr, CH)])


def _make_sc(with_counts):
    mesh = plsc.VectorSubcoreMesh(core_axis_name="c", subcore_axis_name="s",
                                  num_cores=NC, num_subcores=NS)
    out_type = [jax.ShapeDtypeStruct((NC, N_PAD, D), jnp.float32)]
    scratch = [
        pltpu.VMEM_SHARED((N_PAD, D), jnp.float32),
        pltpu.VMEM((2, CH), jnp.int32),
        pltpu.VMEM((CH, D), jnp.float32),
        pltpu.SemaphoreType.DMA,
    ]
    if with_counts:
        out_type.append(jax.ShapeDtypeStruct((NC, N_PAD, 16), jnp.float32))
        scratch.insert(1, pltpu.VMEM_SHARED((N_PAD, 16), jnp.float32))
        scratch.insert(4, pltpu.VMEM((CH, 16), jnp.float32))
    return pl.kernel(
        functools.partial(_sc_body, with_counts),
        out_type=out_type,
        mesh=mesh,
        scratch_types=scratch,
    )


_sc_agg_cnt = _make_sc(True)
_sc_agg = _make_sc(False)


_DN = (((1,), (1,)), ((), ()))


def _tc_in_body(x_ref, wl_ref, wr_ref, b_ref, y_ref, z_ref):
    x = x_ref[...]
    y_ref[...] = lax.dot_general(x, wl_ref[...], _DN,
                                 preferred_element_type=jnp.float32)
    z_ref[...] = lax.dot_general(x, wr_ref[...], _DN,
                                 preferred_element_type=jnp.float32) + b_ref[...]


def _tc_mid_body(acc_ref, cnt_ref, z1_ref, wl_ref, wr_ref, b_ref,
                 y2_ref, z2_ref):
    ssum = acc_ref[0] + acc_ref[1]
    cnt = cnt_ref[0, :, 0:1] + cnt_ref[1, :, 0:1]
    h = ssum / jnp.maximum(cnt, 1.0) + z1_ref[...]
    h = jnp.maximum(h, 0.0)
    y2_ref[...] = lax.dot_general(h, wl_ref[...], _DN,
                                  preferred_element_type=jnp.float32)
    z2_ref[...] = lax.dot_general(h, wr_ref[...], _DN,
                                  preferred_element_type=jnp.float32) + b_ref[...]


def _tc_out_body(acc_ref, cnt_ref, z2_ref, out_ref):
    ssum = acc_ref[0] + acc_ref[1]
    cnt = cnt_ref[0, :, 0:1] + cnt_ref[1, :, 0:1]
    out_ref[...] = ssum / jnp.maximum(cnt, 1.0) + z2_ref[...]


def _row_spec(bm=BM):
    return pl.BlockSpec((bm, D), lambda i: (i, 0))


_W_SPEC = pl.BlockSpec((D, D), lambda i: (0, 0))
_B_SPEC = pl.BlockSpec((1, D), lambda i: (0, 0))
_ACC_SPEC = pl.BlockSpec((NC, BM, D), lambda i: (0, i, 0))
_CNT_SPEC = pl.BlockSpec((NC, BM, 16), lambda i: (0, i, 0))
_GRID = (N_PAD // BM,)

_tc_in = pl.pallas_call(
    _tc_in_body,
    grid=_GRID,
    in_specs=[_row_spec(), _W_SPEC, _W_SPEC, _B_SPEC],
    out_specs=[_row_spec(), _row_spec()],
    out_shape=[jax.ShapeDtypeStruct((N_PAD, D), jnp.float32)] * 2,
)

_tc_mid = pl.pallas_call(
    _tc_mid_body,
    grid=_GRID,
    in_specs=[_ACC_SPEC, _CNT_SPEC, _row_spec(), _W_SPEC, _W_SPEC, _B_SPEC],
    out_specs=[_row_spec(), _row_spec()],
    out_shape=[jax.ShapeDtypeStruct((N_PAD, D), jnp.float32)] * 2,
)

_tc_out = pl.pallas_call(
    _tc_out_body,
    grid=_GRID,
    in_specs=[_ACC_SPEC, _CNT_SPEC, _row_spec()],
    out_specs=_row_spec(),
    out_shape=jax.ShapeDtypeStruct((N_PAD, D), jnp.float32),
)


def kernel(x, edge_index, W_l1, b1, W_r1, W_l2, b2, W_r2):
    x_pad = jnp.pad(x, ((0, N_PAD - N), (0, 0)))
    ei = edge_index.astype(jnp.int32)
    src = jnp.pad(ei[0], (0, E_PAD - E)).reshape(NW, NCHUNK, 1, CH)
    dst = jnp.pad(ei[1], (0, E_PAD - E),
                  constant_values=PAD_DST).reshape(NW, NCHUNK, 1, CH)
    eidx = jnp.concatenate([src, dst], axis=2)
    zmat = jnp.zeros((CH, D), jnp.float32)
    wrow = jnp.arange(NS * NZ * CH, dtype=jnp.int32).reshape(NS, NZ, 1, CH)
    widx = jnp.concatenate([wrow, wrow], axis=2)
    zo16 = jnp.concatenate(
        [jnp.zeros((1, CH, 16), jnp.float32), jnp.ones((1, CH, 16), jnp.float32)])
    b1r = b1.reshape(1, D)
    b2r = b2.reshape(1, D)

    y1, z1 = _tc_in(x_pad, W_l1, W_r1, b1r)
    acc1, cnt = _sc_agg_cnt(y1, eidx, zmat, zo16, widx)
    y2, z2 = _tc_mid(acc1, cnt, z1, W_l2, W_r2, b2r)
    (acc2,) = _sc_agg(y2, eidx, zmat)
    out = _tc_out(acc2, cnt, z2)
    return out[:N]


# speedup vs baseline: 6.0484x; 1.4188x over previous
import functools

import jax
import jax.numpy as jnp
from jax import lax
from jax.experimental import pallas as pl
from jax.experimental.pallas import tpu as pltpu
from jax.experimental.pallas import tpu_sc as plsc

N = 10000
E = 320000
D = 128
N_PAD = 10240
NC, NS = 2, 16
NW = NC * NS
CH = 128
NCHUNK = 79
EW = NCHUNK * CH
E_PAD = NW * EW
PAD_DST = N
RPT = N_PAD // NS
NZ = RPT // CH
BM = 512


def _sc_body(with_counts, ch, nchunk, *refs):
    nz = RPT // ch
    if with_counts:
        (y_hbm, eidx_hbm, zmat_hbm, zo16_hbm, widx_hbm,
         acc_out, cnt_out,
         acc_sh, cnt_sh, idx_v, idx_b, rows_v, rows_b, c16_v, sem) = refs
    else:
        (y_hbm, eidx_hbm, zmat_hbm,
         acc_out,
         acc_sh, idx_v, idx_b, rows_v, rows_b, sem) = refs

    c = lax.axis_index("c")
    s = lax.axis_index("s")
    wid = s * NC + c
    r0 = s * RPT

    pltpu.sync_copy(zmat_hbm.at[pl.ds(0, ch)], rows_v)

    @pl.loop(0, nz)
    def _zero_acc(j):
        pltpu.sync_copy(rows_v, acc_sh.at[pl.ds(r0 + j * ch, ch)])

    if with_counts:
        pltpu.sync_copy(zo16_hbm.at[0, pl.ds(0, ch)], c16_v)

        @pl.loop(0, nz)
        def _zero_cnt(j):
            pltpu.sync_copy(widx_hbm.at[s, j], idx_v)
            pltpu.sync_copy(c16_v, cnt_sh.at[idx_v.at[0]])

        pltpu.sync_copy(zo16_hbm.at[1, pl.ds(0, ch)], c16_v)

    plsc.subcore_barrier()

    @pl.loop(0, nchunk - 1, step=2)
    def _edges(j):
        pltpu.sync_copy(eidx_hbm.at[wid, j], idx_v)
        cp_a = pltpu.async_copy(y_hbm.at[idx_v.at[0]], rows_v, sem)
        pltpu.sync_copy(eidx_hbm.at[wid, j + 1], idx_b)
        cp_b = pltpu.async_copy(y_hbm.at[idx_b.at[0]], rows_b, sem)
        cp_a.wait()
        pltpu.sync_copy(rows_v, acc_sh.at[idx_v.at[1]], add=True)
        if with_counts:
            pltpu.sync_copy(c16_v, cnt_sh.at[idx_v.at[1]], add=True)
        cp_b.wait()
        pltpu.sync_copy(rows_b, acc_sh.at[idx_b.at[1]], add=True)
        if with_counts:
            pltpu.sync_copy(c16_v, cnt_sh.at[idx_b.at[1]], add=True)

    pltpu.sync_copy(eidx_hbm.at[wid, nchunk - 1], idx_v)
    pltpu.async_copy(y_hbm.at[idx_v.at[0]], rows_v, sem).wait()
    pltpu.sync_copy(rows_v, acc_sh.at[idx_v.at[1]], add=True)
    if with_counts:
        pltpu.sync_copy(c16_v, cnt_sh.at[idx_v.at[1]], add=True)

    plsc.subcore_barrier()

    @pl.loop(0, nz)
    def _writeout(j):
        rr = r0 + j * ch
        pltpu.async_copy(acc_sh.at[pl.ds(rr, ch)], rows_v, sem).wait()
        pltpu.sync_copy(rows_v, acc_out.at[c, pl.ds(rr, ch)])

    if with_counts:

        @pl.loop(0, nz)
        def _writeout_cnt(j):
            rr = r0 + j * ch
            pltpu.sync_copy(widx_hbm.at[s, j], idx_v)
            pltpu.async_copy(cnt_sh.at[idx_v.at[0]], c16_v, sem).wait()
            pltpu.sync_copy(c16_v, cnt_out.at[c, pl.ds(rr, ch)])


def _make_sc(with_counts, ch, nchunk):
    mesh = plsc.VectorSubcoreMesh(core_axis_name="c", subcore_axis_name="s",
                                  num_cores=NC, num_subcores=NS)
    out_type = [jax.ShapeDtypeStruct((NC, N_PAD, D), jnp.float32)]
    scratch = [
        pltpu.VMEM_SHARED((N_PAD, D), jnp.float32),
        pltpu.VMEM((2, ch), jnp.int32),
        pltpu.VMEM((2, ch), jnp.int32),
        pltpu.VMEM((ch, D), jnp.float32),
        pltpu.VMEM((ch, D), jnp.float32),
        pltpu.SemaphoreType.DMA,
    ]
    if with_counts:
        out_type.append(jax.ShapeDtypeStruct((NC, N_PAD, 16), jnp.float32))
        scratch.insert(1, pltpu.VMEM_SHARED((N_PAD, 16), jnp.float32))
        scratch.insert(6, pltpu.VMEM((ch, 16), jnp.float32))
    return pl.kernel(
        functools.partial(_sc_body, with_counts, ch, nchunk),
        out_type=out_type,
        mesh=mesh,
        scratch_types=scratch,
    )


CH1 = 80
NCHUNK1 = E // NW // CH1
_sc_agg_cnt = _make_sc(True, CH1, NCHUNK1)
_sc_agg = _make_sc(False, CH, NCHUNK)


_DN = (((1,), (1,)), ((), ()))


def _tc_in_body(x_ref, wl_ref, wr_ref, b_ref, y_ref, z_ref):
    x = x_ref[...]
    y_ref[...] = lax.dot_general(x, wl_ref[...], _DN,
                                 preferred_element_type=jnp.float32)
    z_ref[...] = lax.dot_general(x, wr_ref[...], _DN,
                                 preferred_element_type=jnp.float32) + b_ref[...]


def _tc_mid_body(acc_ref, cnt_ref, z1_ref, wl_ref, wr_ref, b_ref,
                 y2_ref, z2_ref):
    ssum = acc_ref[0] + acc_ref[1]
    cnt = cnt_ref[0, :, 0:1] + cnt_ref[1, :, 0:1]
    h = ssum / jnp.maximum(cnt, 1.0) + z1_ref[...]
    h = jnp.maximum(h, 0.0)
    y2_ref[...] = lax.dot_general(h, wl_ref[...], _DN,
                                  preferred_element_type=jnp.float32)
    z2_ref[...] = lax.dot_general(h, wr_ref[...], _DN,
                                  preferred_element_type=jnp.float32) + b_ref[...]


def _tc_out_body(acc_ref, cnt_ref, z2_ref, out_ref):
    ssum = acc_ref[0] + acc_ref[1]
    cnt = cnt_ref[0, :, 0:1] + cnt_ref[1, :, 0:1]
    out_ref[...] = ssum / jnp.maximum(cnt, 1.0) + z2_ref[...]


def _row_spec(bm=BM):
    return pl.BlockSpec((bm, D), lambda i: (i, 0))


_W_SPEC = pl.BlockSpec((D, D), lambda i: (0, 0))
_B_SPEC = pl.BlockSpec((1, D), lambda i: (0, 0))
_ACC_SPEC = pl.BlockSpec((NC, BM, D), lambda i: (0, i, 0))
_CNT_SPEC = pl.BlockSpec((NC, BM, 16), lambda i: (0, i, 0))
_GRID = (N_PAD // BM,)

_tc_in = pl.pallas_call(
    _tc_in_body,
    grid=_GRID,
    in_specs=[_row_spec(), _W_SPEC, _W_SPEC, _B_SPEC],
    out_specs=[_row_spec(), _row_spec()],
    out_shape=[jax.ShapeDtypeStruct((N_PAD, D), jnp.float32)] * 2,
)

_tc_mid = pl.pallas_call(
    _tc_mid_body,
    grid=_GRID,
    in_specs=[_ACC_SPEC, _CNT_SPEC, _row_spec(), _W_SPEC, _W_SPEC, _B_SPEC],
    out_specs=[_row_spec(), _row_spec()],
    out_shape=[jax.ShapeDtypeStruct((N_PAD, D), jnp.float32)] * 2,
)

_tc_out = pl.pallas_call(
    _tc_out_body,
    grid=_GRID,
    in_specs=[_ACC_SPEC, _CNT_SPEC, _row_spec()],
    out_specs=_row_spec(),
    out_shape=jax.ShapeDtypeStruct((N_PAD, D), jnp.float32),
)


def kernel(x, edge_index, W_l1, b1, W_r1, W_l2, b2, W_r2):
    x_pad = jnp.pad(x, ((0, N_PAD - N), (0, 0)))
    ei = edge_index.astype(jnp.int32)
    src = jnp.pad(ei[0], (0, E_PAD - E)).reshape(NW, NCHUNK, 1, CH)
    dst = jnp.pad(ei[1], (0, E_PAD - E),
                  constant_values=PAD_DST).reshape(NW, NCHUNK, 1, CH)
    eidx = jnp.concatenate([src, dst], axis=2)
    src1 = ei[0].reshape(NW, NCHUNK1, 1, CH1)
    dst1 = ei[1].reshape(NW, NCHUNK1, 1, CH1)
    eidx1 = jnp.concatenate([src1, dst1], axis=2)
    zmat = jnp.zeros((CH, D), jnp.float32)
    nz1 = RPT // CH1
    wrow = jnp.arange(NS * RPT, dtype=jnp.int32).reshape(NS, nz1, 1, CH1)
    widx = jnp.concatenate([wrow, wrow], axis=2)
    zo16 = jnp.concatenate(
        [jnp.zeros((1, CH, 16), jnp.float32), jnp.ones((1, CH, 16), jnp.float32)])
    b1r = b1.reshape(1, D)
    b2r = b2.reshape(1, D)

    y1, z1 = _tc_in(x_pad, W_l1, W_r1, b1r)
    acc1, cnt = _sc_agg_cnt(y1, eidx1, zmat, zo16, widx)
    y2, z2 = _tc_mid(acc1, cnt, z1, W_l2, W_r2, b2r)
    (acc2,) = _sc_agg(y2, eidx, zmat)
    out = _tc_out(acc2, cnt, z2)
    return out[:N]


# speedup vs baseline: 7.7911x; 1.2881x over previous
import functools

import jax
import jax.numpy as jnp
from jax import lax
from jax.experimental import pallas as pl
from jax.experimental.pallas import tpu as pltpu
from jax.experimental.pallas import tpu_sc as plsc

N = 10000
E = 320000
D = 128
N_PAD = 10240
NC, NS = 2, 16
NW = NC * NS
CH = 128
NCHUNK = 79
EW = NCHUNK * CH
E_PAD = NW * EW
PAD_DST = N
RPT = N_PAD // NS
NZ = RPT // CH
BM = 512


def _sc_body(with_counts, ch, nchunk, *refs):
    nz = RPT // ch
    if with_counts:
        (y_hbm, eidx_hbm, zmat_hbm, zo16_hbm, widx_hbm,
         acc_out, cnt_out,
         acc_sh, cnt_sh, idx_v, idx_b, rows_v, rows_b, c16_v, sem) = refs
    else:
        (y_hbm, eidx_hbm, zmat_hbm,
         acc_out,
         acc_sh, idx_v, idx_b, rows_v, rows_b, sem) = refs

    c = lax.axis_index("c")
    s = lax.axis_index("s")
    wid = s * NC + c
    r0 = s * RPT

    pltpu.sync_copy(zmat_hbm.at[pl.ds(0, ch)], rows_v)

    @pl.loop(0, nz)
    def _zero_acc(j):
        pltpu.sync_copy(rows_v, acc_sh.at[pl.ds(r0 + j * ch, ch)])

    if with_counts:
        pltpu.sync_copy(zo16_hbm.at[0, pl.ds(0, ch)], c16_v)

        @pl.loop(0, nz)
        def _zero_cnt(j):
            pltpu.sync_copy(widx_hbm.at[s, j], idx_v)
            pltpu.sync_copy(c16_v, cnt_sh.at[idx_v.at[0]])

        pltpu.sync_copy(zo16_hbm.at[1, pl.ds(0, ch)], c16_v)

    plsc.subcore_barrier()

    @pl.loop(0, nchunk - 1, step=2)
    def _edges(j):
        pltpu.sync_copy(eidx_hbm.at[wid, j], idx_v)
        cp_a = pltpu.async_copy(y_hbm.at[idx_v.at[0]], rows_v, sem)
        pltpu.sync_copy(eidx_hbm.at[wid, j + 1], idx_b)
        cp_b = pltpu.async_copy(y_hbm.at[idx_b.at[0]], rows_b, sem)
        cp_a.wait()
        pltpu.sync_copy(rows_v, acc_sh.at[idx_v.at[1]], add=True)
        if with_counts:
            pltpu.sync_copy(c16_v, cnt_sh.at[idx_v.at[1]], add=True)
        cp_b.wait()
        pltpu.sync_copy(rows_b, acc_sh.at[idx_b.at[1]], add=True)
        if with_counts:
            pltpu.sync_copy(c16_v, cnt_sh.at[idx_b.at[1]], add=True)

    pltpu.sync_copy(eidx_hbm.at[wid, nchunk - 1], idx_v)
    pltpu.async_copy(y_hbm.at[idx_v.at[0]], rows_v, sem).wait()
    pltpu.sync_copy(rows_v, acc_sh.at[idx_v.at[1]], add=True)
    if with_counts:
        pltpu.sync_copy(c16_v, cnt_sh.at[idx_v.at[1]], add=True)

    plsc.subcore_barrier()

    @pl.loop(0, nz)
    def _writeout(j):
        rr = r0 + j * ch
        pltpu.async_copy(acc_sh.at[pl.ds(rr, ch)], rows_v, sem).wait()
        pltpu.sync_copy(rows_v, acc_out.at[c, pl.ds(rr, ch)])

    if with_counts:

        @pl.loop(0, nz)
        def _writeout_cnt(j):
            rr = r0 + j * ch
            pltpu.sync_copy(widx_hbm.at[s, j], idx_v)
            pltpu.async_copy(cnt_sh.at[idx_v.at[0]], c16_v, sem).wait()
            pltpu.sync_copy(c16_v, cnt_out.at[c, pl.ds(rr, ch)])


def _make_sc(with_counts, ch, nchunk):
    mesh = plsc.VectorSubcoreMesh(core_axis_name="c", subcore_axis_name="s",
                                  num_cores=NC, num_subcores=NS)
    out_type = [jax.ShapeDtypeStruct((NC, N_PAD, D), jnp.float32)]
    scratch = [
        pltpu.VMEM_SHARED((N_PAD, D), jnp.float32),
        pltpu.VMEM((2, ch), jnp.int32),
        pltpu.VMEM((2, ch), jnp.int32),
        pltpu.VMEM((ch, D), jnp.float32),
        pltpu.VMEM((ch, D), jnp.float32),
        pltpu.SemaphoreType.DMA,
    ]
    if with_counts:
        out_type.append(jax.ShapeDtypeStruct((NC, N_PAD, 16), jnp.float32))
        scratch.insert(1, pltpu.VMEM_SHARED((N_PAD, 16), jnp.float32))
        scratch.insert(6, pltpu.VMEM((ch, 16), jnp.float32))
    return pl.kernel(
        functools.partial(_sc_body, with_counts, ch, nchunk),
        out_type=out_type,
        mesh=mesh,
        scratch_types=scratch,
    )


CH1 = 80
NCHUNK1 = E // NW // CH1
_sc_agg_cnt = _make_sc(True, CH1, NCHUNK1)
_sc_agg = _make_sc(False, CH1, NCHUNK1)


_DN = (((1,), (1,)), ((), ()))


def _tc_in_body(x_ref, wl_ref, wr_ref, b_ref, y_ref, z_ref):
    x = x_ref[...]
    y_ref[...] = lax.dot_general(x, wl_ref[...], _DN,
                                 preferred_element_type=jnp.float32)
    z_ref[...] = lax.dot_general(x, wr_ref[...], _DN,
                                 preferred_element_type=jnp.float32) + b_ref[...]


def _tc_mid_body(acc_ref, cnt_ref, z1_ref, wl_ref, wr_ref, b_ref,
                 y2_ref, z2_ref):
    ssum = acc_ref[0] + acc_ref[1]
    cnt = cnt_ref[0, :, 0:1] + cnt_ref[1, :, 0:1]
    h = ssum / jnp.maximum(cnt, 1.0) + z1_ref[...]
    h = jnp.maximum(h, 0.0)
    y2_ref[...] = lax.dot_general(h, wl_ref[...], _DN,
                                  preferred_element_type=jnp.float32)
    z2_ref[...] = lax.dot_general(h, wr_ref[...], _DN,
                                  preferred_element_type=jnp.float32) + b_ref[...]


def _tc_out_body(acc_ref, cnt_ref, z2_ref, out_ref):
    ssum = acc_ref[0] + acc_ref[1]
    cnt = cnt_ref[0, :, 0:1] + cnt_ref[1, :, 0:1]
    out_ref[...] = ssum / jnp.maximum(cnt, 1.0) + z2_ref[...]


def _row_spec(bm=BM):
    return pl.BlockSpec((bm, D), lambda i: (i, 0))


_W_SPEC = pl.BlockSpec((D, D), lambda i: (0, 0))
_B_SPEC = pl.BlockSpec((1, D), lambda i: (0, 0))
_ACC_SPEC = pl.BlockSpec((NC, BM, D), lambda i: (0, i, 0))
_CNT_SPEC = pl.BlockSpec((NC, BM, 16), lambda i: (0, i, 0))
_GRID = (N_PAD // BM,)

_tc_in = pl.pallas_call(
    _tc_in_body,
    grid=_GRID,
    in_specs=[_row_spec(), _W_SPEC, _W_SPEC, _B_SPEC],
    out_specs=[_row_spec(), _row_spec()],
    out_shape=[jax.ShapeDtypeStruct((N_PAD, D), jnp.float32)] * 2,
)

_tc_mid = pl.pallas_call(
    _tc_mid_body,
    grid=_GRID,
    in_specs=[_ACC_SPEC, _CNT_SPEC, _row_spec(), _W_SPEC, _W_SPEC, _B_SPEC],
    out_specs=[_row_spec(), _row_spec()],
    out_shape=[jax.ShapeDtypeStruct((N_PAD, D), jnp.float32)] * 2,
)

_tc_out = pl.pallas_call(
    _tc_out_body,
    grid=_GRID,
    in_specs=[_ACC_SPEC, _CNT_SPEC, _row_spec()],
    out_specs=_row_spec(),
    out_shape=jax.ShapeDtypeStruct((N_PAD, D), jnp.float32),
)


def kernel(x, edge_index, W_l1, b1, W_r1, W_l2, b2, W_r2):
    x_pad = jnp.pad(x, ((0, N_PAD - N), (0, 0)))
    ei = edge_index.astype(jnp.int32)
    src1 = ei[0].reshape(NW, NCHUNK1, 1, CH1)
    dst1 = ei[1].reshape(NW, NCHUNK1, 1, CH1)
    eidx1 = jnp.concatenate([src1, dst1], axis=2)
    zmat = jnp.zeros((CH, D), jnp.float32)
    nz1 = RPT // CH1
    wrow = jnp.arange(NS * RPT, dtype=jnp.int32).reshape(NS, nz1, 1, CH1)
    widx = jnp.concatenate([wrow, wrow], axis=2)
    zo16 = jnp.concatenate(
        [jnp.zeros((1, CH, 16), jnp.float32), jnp.ones((1, CH, 16), jnp.float32)])
    b1r = b1.reshape(1, D)
    b2r = b2.reshape(1, D)

    y1, z1 = _tc_in(x_pad, W_l1, W_r1, b1r)
    acc1, cnt = _sc_agg_cnt(y1, eidx1, zmat, zo16, widx)
    y2, z2 = _tc_mid(acc1, cnt, z1, W_l2, W_r2, b2r)
    (acc2,) = _sc_agg(y2, eidx1, zmat)
    out = _tc_out(acc2, cnt, z2)
    return out[:N]


# speedup vs baseline: 8.8803x; 1.1398x over previous
import functools

import jax
import jax.numpy as jnp
from jax import lax
from jax.experimental import pallas as pl
from jax.experimental.pallas import tpu as pltpu
from jax.experimental.pallas import tpu_sc as plsc

N = 10000
E = 320000
D = 128
N_PAD = 10240
NC, NS = 2, 16
NW = NC * NS
CH = 128
NCHUNK = 79
EW = NCHUNK * CH
E_PAD = NW * EW
PAD_DST = N
RPT = N_PAD // NS
NZ = RPT // CH
BM = 512


def _sc_body(with_counts, ch, nchunk, *refs):
    nz = RPT // ch
    if with_counts:
        (y_hbm, eidx_hbm, zmat_hbm, zo16_hbm, widx_hbm,
         acc_out, cnt_out,
         acc_sh, cnt_sh, idx_v, idx_b, rows_v, rows_b, c16_v,
         sem, sem_b) = refs
    else:
        (y_hbm, eidx_hbm, zmat_hbm,
         acc_out,
         acc_sh, idx_v, idx_b, rows_v, rows_b, sem, sem_b) = refs

    c = lax.axis_index("c")
    s = lax.axis_index("s")
    wid = s * NC + c
    r0 = s * RPT

    pltpu.sync_copy(zmat_hbm.at[pl.ds(0, ch)], rows_v)

    @pl.loop(0, nz)
    def _zero_acc(j):
        pltpu.sync_copy(rows_v, acc_sh.at[pl.ds(r0 + j * ch, ch)])

    if with_counts:
        pltpu.sync_copy(zo16_hbm.at[0, pl.ds(0, ch)], c16_v)

        @pl.loop(0, nz)
        def _zero_cnt(j):
            pltpu.sync_copy(widx_hbm.at[s, j], idx_v)
            pltpu.sync_copy(c16_v, cnt_sh.at[idx_v.at[0]])

        pltpu.sync_copy(zo16_hbm.at[1, pl.ds(0, ch)], c16_v)

    plsc.subcore_barrier()

    def _drain(buf, dsem):
        pltpu.make_async_copy(y_hbm.at[pl.ds(0, ch)], buf, dsem).wait()

    pltpu.sync_copy(eidx_hbm.at[wid, 0], idx_v)
    pltpu.async_copy(y_hbm.at[idx_v.at[0]], rows_v, sem)

    @pl.loop(0, (nchunk - 1) // 2)
    def _edges(p):
        b = 2 * p + 1
        pltpu.sync_copy(eidx_hbm.at[wid, b], idx_b)
        pltpu.async_copy(y_hbm.at[idx_b.at[0]], rows_b, sem_b)
        _drain(rows_v, sem)
        pltpu.sync_copy(rows_v, acc_sh.at[idx_v.at[1]], add=True)
        if with_counts:
            pltpu.sync_copy(c16_v, cnt_sh.at[idx_v.at[1]], add=True)
        pltpu.sync_copy(eidx_hbm.at[wid, b + 1], idx_v)
        pltpu.async_copy(y_hbm.at[idx_v.at[0]], rows_v, sem)
        _drain(rows_b, sem_b)
        pltpu.sync_copy(rows_b, acc_sh.at[idx_b.at[1]], add=True)
        if with_counts:
            pltpu.sync_copy(c16_v, cnt_sh.at[idx_b.at[1]], add=True)

    _drain(rows_v, sem)
    pltpu.sync_copy(rows_v, acc_sh.at[idx_v.at[1]], add=True)
    if with_counts:
        pltpu.sync_copy(c16_v, cnt_sh.at[idx_v.at[1]], add=True)

    plsc.subcore_barrier()

    @pl.loop(0, nz)
    def _writeout(j):
        rr = r0 + j * ch
        pltpu.async_copy(acc_sh.at[pl.ds(rr, ch)], rows_v, sem).wait()
        pltpu.sync_copy(rows_v, acc_out.at[c, pl.ds(rr, ch)])

    if with_counts:

        @pl.loop(0, nz)
        def _writeout_cnt(j):
            rr = r0 + j * ch
            pltpu.sync_copy(widx_hbm.at[s, j], idx_v)
            pltpu.async_copy(cnt_sh.at[idx_v.at[0]], c16_v, sem).wait()
            pltpu.sync_copy(c16_v, cnt_out.at[c, pl.ds(rr, ch)])


def _make_sc(with_counts, ch, nchunk):
    mesh = plsc.VectorSubcoreMesh(core_axis_name="c", subcore_axis_name="s",
                                  num_cores=NC, num_subcores=NS)
    out_type = [jax.ShapeDtypeStruct((NC, N_PAD, D), jnp.float32)]
    scratch = [
        pltpu.VMEM_SHARED((N_PAD, D), jnp.float32),
        pltpu.VMEM((2, ch), jnp.int32),
        pltpu.VMEM((2, ch), jnp.int32),
        pltpu.VMEM((ch, D), jnp.float32),
        pltpu.VMEM((ch, D), jnp.float32),
        pltpu.SemaphoreType.DMA,
        pltpu.SemaphoreType.DMA,
    ]
    if with_counts:
        out_type.append(jax.ShapeDtypeStruct((NC, N_PAD, 16), jnp.float32))
        scratch.insert(1, pltpu.VMEM_SHARED((N_PAD, 16), jnp.float32))
        scratch.insert(6, pltpu.VMEM((ch, 16), jnp.float32))
    return pl.kernel(
        functools.partial(_sc_body, with_counts, ch, nchunk),
        out_type=out_type,
        mesh=mesh,
        scratch_types=scratch,
    )


CH1 = 80
NCHUNK1 = E // NW // CH1
_sc_agg_cnt = _make_sc(True, CH1, NCHUNK1)
_sc_agg = _make_sc(False, CH1, NCHUNK1)


_DN = (((1,), (1,)), ((), ()))


def _tc_in_body(x_ref, wl_ref, wr_ref, b_ref, y_ref, z_ref):
    x = x_ref[...]
    y_ref[...] = lax.dot_general(x, wl_ref[...], _DN,
                                 preferred_element_type=jnp.float32)
    z_ref[...] = lax.dot_general(x, wr_ref[...], _DN,
                                 preferred_element_type=jnp.float32) + b_ref[...]


def _tc_mid_body(acc_ref, cnt_ref, z1_ref, wl_ref, wr_ref, b_ref,
                 y2_ref, z2_ref):
    ssum = acc_ref[0] + acc_ref[1]
    cnt = cnt_ref[0, :, 0:1] + cnt_ref[1, :, 0:1]
    h = ssum / jnp.maximum(cnt, 1.0) + z1_ref[...]
    h = jnp.maximum(h, 0.0)
    y2_ref[...] = lax.dot_general(h, wl_ref[...], _DN,
                                  preferred_element_type=jnp.float32)
    z2_ref[...] = lax.dot_general(h, wr_ref[...], _DN,
                                  preferred_element_type=jnp.float32) + b_ref[...]


def _tc_out_body(acc_ref, cnt_ref, z2_ref, out_ref):
    ssum = acc_ref[0] + acc_ref[1]
    cnt = cnt_ref[0, :, 0:1] + cnt_ref[1, :, 0:1]
    out_ref[...] = ssum / jnp.maximum(cnt, 1.0) + z2_ref[...]


def _row_spec(bm=BM):
    return pl.BlockSpec((bm, D), lambda i: (i, 0))


_W_SPEC = pl.BlockSpec((D, D), lambda i: (0, 0))
_B_SPEC = pl.BlockSpec((1, D), lambda i: (0, 0))
_ACC_SPEC = pl.BlockSpec((NC, BM, D), lambda i: (0, i, 0))
_CNT_SPEC = pl.BlockSpec((NC, BM, 16), lambda i: (0, i, 0))
_GRID = (N_PAD // BM,)

_tc_in = pl.pallas_call(
    _tc_in_body,
    grid=_GRID,
    in_specs=[_row_spec(), _W_SPEC, _W_SPEC, _B_SPEC],
    out_specs=[_row_spec(), _row_spec()],
    out_shape=[jax.ShapeDtypeStruct((N_PAD, D), jnp.float32)] * 2,
)

_tc_mid = pl.pallas_call(
    _tc_mid_body,
    grid=_GRID,
    in_specs=[_ACC_SPEC, _CNT_SPEC, _row_spec(), _W_SPEC, _W_SPEC, _B_SPEC],
    out_specs=[_row_spec(), _row_spec()],
    out_shape=[jax.ShapeDtypeStruct((N_PAD, D), jnp.float32)] * 2,
)

_tc_out = pl.pallas_call(
    _tc_out_body,
    grid=_GRID,
    in_specs=[_ACC_SPEC, _CNT_SPEC, _row_spec()],
    out_specs=_row_spec(),
    out_shape=jax.ShapeDtypeStruct((N_PAD, D), jnp.float32),
)


def kernel(x, edge_index, W_l1, b1, W_r1, W_l2, b2, W_r2):
    x_pad = jnp.pad(x, ((0, N_PAD - N), (0, 0)))
    ei = edge_index.astype(jnp.int32)
    src1 = ei[0].reshape(NW, NCHUNK1, 1, CH1)
    dst1 = ei[1].reshape(NW, NCHUNK1, 1, CH1)
    eidx1 = jnp.concatenate([src1, dst1], axis=2)
    zmat = jnp.zeros((CH, D), jnp.float32)
    nz1 = RPT // CH1
    wrow = jnp.arange(NS * RPT, dtype=jnp.int32).reshape(NS, nz1, 1, CH1)
    widx = jnp.concatenate([wrow, wrow], axis=2)
    zo16 = jnp.concatenate(
        [jnp.zeros((1, CH, 16), jnp.float32), jnp.ones((1, CH, 16), jnp.float32)])
    b1r = b1.reshape(1, D)
    b2r = b2.reshape(1, D)

    y1, z1 = _tc_in(x_pad, W_l1, W_r1, b1r)
    acc1, cnt = _sc_agg_cnt(y1, eidx1, zmat, zo16, widx)
    y2, z2 = _tc_mid(acc1, cnt, z1, W_l2, W_r2, b2r)
    (acc2,) = _sc_agg(y2, eidx1, zmat)
    out = _tc_out(acc2, cnt, z2)
    return out[:N]


# speedup vs baseline: 8.8867x; 1.0007x over previous
import functools

import jax
import jax.numpy as jnp
from jax import lax
from jax.experimental import pallas as pl
from jax.experimental.pallas import tpu as pltpu
from jax.experimental.pallas import tpu_sc as plsc

N = 10000
E = 320000
D = 128
N_PAD = 10240
NC, NS = 2, 16
NW = NC * NS
CH = 128
NCHUNK = 79
EW = NCHUNK * CH
E_PAD = NW * EW
PAD_DST = N
RPT = N_PAD // NS
NZ = RPT // CH
BM = 512


def _sc_body(with_counts, ch, nchunk, *refs):
    nz = RPT // ch
    if with_counts:
        (y_hbm, eidx_hbm, zmat_hbm, zo16_hbm, widx_hbm,
         acc_out, cnt_out,
         acc_sh, cnt_sh, p0_v, p1_v, rows_v, rows_b, c16_v,
         sem, sem_b) = refs
    else:
        (y_hbm, eidx_hbm, zmat_hbm,
         acc_out,
         acc_sh, p0_v, p1_v, rows_v, rows_b, sem, sem_b) = refs

    c = lax.axis_index("c")
    s = lax.axis_index("s")
    wid = s * NC + c
    r0 = s * RPT

    pltpu.sync_copy(zmat_hbm.at[pl.ds(0, ch)], rows_v)

    @pl.loop(0, nz)
    def _zero_acc(j):
        pltpu.sync_copy(rows_v, acc_sh.at[pl.ds(r0 + j * ch, ch)])

    if with_counts:
        pltpu.sync_copy(zo16_hbm.at[0, pl.ds(0, ch)], c16_v)

        @pl.loop(0, nz)
        def _zero_cnt(j):
            pltpu.sync_copy(widx_hbm.at[s, j], p0_v.at[0])
            pltpu.sync_copy(c16_v, cnt_sh.at[p0_v.at[0, 0]])

        pltpu.sync_copy(zo16_hbm.at[1, pl.ds(0, ch)], c16_v)

    plsc.subcore_barrier()

    def _drain(buf, dsem):
        pltpu.make_async_copy(y_hbm.at[pl.ds(0, ch)], buf, dsem).wait()

    def _scat(buf, didx):
        pltpu.sync_copy(buf, acc_sh.at[didx], add=True)
        if with_counts:
            pltpu.sync_copy(c16_v, cnt_sh.at[didx], add=True)

    pltpu.sync_copy(eidx_hbm.at[wid, pl.ds(0, 2)], p0_v)
    pltpu.async_copy(y_hbm.at[p0_v.at[0, 0]], rows_v, sem)

    @pl.loop(0, (nchunk - 1) // 4)
    def _edges(q):
        k = 4 * q
        pltpu.async_copy(y_hbm.at[p0_v.at[1, 0]], rows_b, sem_b)
        _drain(rows_v, sem)
        _scat(rows_v, p0_v.at[0, 1])
        pltpu.sync_copy(eidx_hbm.at[wid, pl.ds(k + 2, 2)], p1_v)
        pltpu.async_copy(y_hbm.at[p1_v.at[0, 0]], rows_v, sem)
        _drain(rows_b, sem_b)
        _scat(rows_b, p0_v.at[1, 1])
        pltpu.async_copy(y_hbm.at[p1_v.at[1, 0]], rows_b, sem_b)
        _drain(rows_v, sem)
        _scat(rows_v, p1_v.at[0, 1])
        pltpu.sync_copy(eidx_hbm.at[wid, pl.ds(k + 4, 2)], p0_v)
        pltpu.async_copy(y_hbm.at[p0_v.at[0, 0]], rows_v, sem)
        _drain(rows_b, sem_b)
        _scat(rows_b, p1_v.at[1, 1])

    _drain(rows_v, sem)
    _scat(rows_v, p0_v.at[0, 1])

    plsc.subcore_barrier()

    @pl.loop(0, nz)
    def _writeout(j):
        rr = r0 + j * ch
        pltpu.async_copy(acc_sh.at[pl.ds(rr, ch)], rows_v, sem).wait()
        pltpu.sync_copy(rows_v, acc_out.at[c, pl.ds(rr, ch)])

    if with_counts:

        @pl.loop(0, nz)
        def _writeout_cnt(j):
            rr = r0 + j * ch
            pltpu.sync_copy(widx_hbm.at[s, j], p0_v.at[0])
            pltpu.async_copy(cnt_sh.at[p0_v.at[0, 0]], c16_v, sem).wait()
            pltpu.sync_copy(c16_v, cnt_out.at[c, pl.ds(rr, ch)])


def _make_sc(with_counts, ch, nchunk):
    mesh = plsc.VectorSubcoreMesh(core_axis_name="c", subcore_axis_name="s",
                                  num_cores=NC, num_subcores=NS)
    out_type = [jax.ShapeDtypeStruct((NC, N_PAD, D), jnp.float32)]
    scratch = [
        pltpu.VMEM_SHARED((N_PAD, D), jnp.float32),
        pltpu.VMEM((2, 2, ch), jnp.int32),
        pltpu.VMEM((2, 2, ch), jnp.int32),
        pltpu.VMEM((ch, D), jnp.float32),
        pltpu.VMEM((ch, D), jnp.float32),
        pltpu.SemaphoreType.DMA,
        pltpu.SemaphoreType.DMA,
    ]
    if with_counts:
        out_type.append(jax.ShapeDtypeStruct((NC, N_PAD, 16), jnp.float32))
        scratch.insert(1, pltpu.VMEM_SHARED((N_PAD, 16), jnp.float32))
        scratch.insert(6, pltpu.VMEM((ch, 16), jnp.float32))
    return pl.kernel(
        functools.partial(_sc_body, with_counts, ch, nchunk),
        out_type=out_type,
        mesh=mesh,
        scratch_types=scratch,
    )


CH1 = 80
NCHUNK1 = E // NW // CH1
_sc_agg_cnt = _make_sc(True, CH1, NCHUNK1)
_sc_agg = _make_sc(False, CH1, NCHUNK1)


_DN = (((1,), (1,)), ((), ()))


def _tc_in_body(x_ref, wl_ref, wr_ref, b_ref, y_ref, z_ref):
    x = x_ref[...]
    y_ref[...] = lax.dot_general(x, wl_ref[...], _DN,
                                 preferred_element_type=jnp.float32)
    z_ref[...] = lax.dot_general(x, wr_ref[...], _DN,
                                 preferred_element_type=jnp.float32) + b_ref[...]


def _tc_mid_body(acc_ref, cnt_ref, z1_ref, wl_ref, wr_ref, b_ref,
                 y2_ref, z2_ref):
    ssum = acc_ref[0] + acc_ref[1]
    cnt = cnt_ref[0, :, 0:1] + cnt_ref[1, :, 0:1]
    h = ssum / jnp.maximum(cnt, 1.0) + z1_ref[...]
    h = jnp.maximum(h, 0.0)
    y2_ref[...] = lax.dot_general(h, wl_ref[...], _DN,
                                  preferred_element_type=jnp.float32)
    z2_ref[...] = lax.dot_general(h, wr_ref[...], _DN,
                                  preferred_element_type=jnp.float32) + b_ref[...]


def _tc_out_body(acc_ref, cnt_ref, z2_ref, out_ref):
    ssum = acc_ref[0] + acc_ref[1]
    cnt = cnt_ref[0, :, 0:1] + cnt_ref[1, :, 0:1]
    out_ref[...] = ssum / jnp.maximum(cnt, 1.0) + z2_ref[...]


def _row_spec(bm=BM):
    return pl.BlockSpec((bm, D), lambda i: (i, 0))


_W_SPEC = pl.BlockSpec((D, D), lambda i: (0, 0))
_B_SPEC = pl.BlockSpec((1, D), lambda i: (0, 0))
_ACC_SPEC = pl.BlockSpec((NC, BM, D), lambda i: (0, i, 0))
_CNT_SPEC = pl.BlockSpec((NC, BM, 16), lambda i: (0, i, 0))
_GRID = (N_PAD // BM,)

_tc_in = pl.pallas_call(
    _tc_in_body,
    grid=_GRID,
    in_specs=[_row_spec(), _W_SPEC, _W_SPEC, _B_SPEC],
    out_specs=[_row_spec(), _row_spec()],
    out_shape=[jax.ShapeDtypeStruct((N_PAD, D), jnp.float32)] * 2,
)

_tc_mid = pl.pallas_call(
    _tc_mid_body,
    grid=_GRID,
    in_specs=[_ACC_SPEC, _CNT_SPEC, _row_spec(), _W_SPEC, _W_SPEC, _B_SPEC],
    out_specs=[_row_spec(), _row_spec()],
    out_shape=[jax.ShapeDtypeStruct((N_PAD, D), jnp.float32)] * 2,
)

_tc_out = pl.pallas_call(
    _tc_out_body,
    grid=_GRID,
    in_specs=[_ACC_SPEC, _CNT_SPEC, _row_spec()],
    out_specs=_row_spec(),
    out_shape=jax.ShapeDtypeStruct((N_PAD, D), jnp.float32),
)


def kernel(x, edge_index, W_l1, b1, W_r1, W_l2, b2, W_r2):
    x_pad = jnp.pad(x, ((0, N_PAD - N), (0, 0)))
    ei = edge_index.astype(jnp.int32)
    src1 = ei[0].reshape(NW, NCHUNK1, 1, CH1)
    dst1 = ei[1].reshape(NW, NCHUNK1, 1, CH1)
    eidx1 = jnp.concatenate([src1, dst1], axis=2)
    eidx1 = jnp.pad(eidx1, ((0, 0), (0, 1), (0, 0), (0, 0)))
    zmat = jnp.zeros((CH, D), jnp.float32)
    nz1 = RPT // CH1
    wrow = jnp.arange(NS * RPT, dtype=jnp.int32).reshape(NS, nz1, 1, CH1)
    widx = jnp.concatenate([wrow, wrow], axis=2)
    zo16 = jnp.concatenate(
        [jnp.zeros((1, CH, 16), jnp.float32), jnp.ones((1, CH, 16), jnp.float32)])
    b1r = b1.reshape(1, D)
    b2r = b2.reshape(1, D)

    y1, z1 = _tc_in(x_pad, W_l1, W_r1, b1r)
    acc1, cnt = _sc_agg_cnt(y1, eidx1, zmat, zo16, widx)
    y2, z2 = _tc_mid(acc1, cnt, z1, W_l2, W_r2, b2r)
    (acc2,) = _sc_agg(y2, eidx1, zmat)
    out = _tc_out(acc2, cnt, z2)
    return out[:N]
